# baseline (speedup 1.0000x reference)
import jax
import jax.numpy as jnp
from jax import lax
from jax.experimental import pallas as pl
from jax.experimental.pallas import tpu as pltpu
from jax.experimental.pallas import tpu_sc as plsc

N = 10000
E = 320000
D = 128
DH = D // 2

NC = 2
NS = 16
CHUNK = 128
CPT = 160
E_PAD = NS * CHUNK * CPT
N_ACC = 10240
ZR = N_ACC // NS
DR = 632
DR_LAST = N - 15 * DR

RB = 2000
NBLK = N // RB


def _seg_sum_kernel(h_hbm, src_hbm, dst_hbm, z_hbm, out_hbm,
                    src_v, dst_v, r0, r1, r2, r3, acc_sh,
                    g0, g1, g2, g3, s0, s1, s2, s3):
    rows = [r0, r1, r2, r3]
    gsem = [g0, g1, g2, g3]
    ssem = [s0, s1, s2, s3]
    cid = lax.axis_index("c")
    sid = lax.axis_index("s")

    pltpu.sync_copy(z_hbm.at[pl.ds(sid * ZR, ZR)], acc_sh.at[pl.ds(sid * ZR, ZR)])
    plsc.subcore_barrier()

    pltpu.sync_copy(src_hbm.at[pl.ds((cid * NS + sid) * CPT, CPT)], src_v)
    pltpu.sync_copy(dst_hbm.at[pl.ds(sid * CPT, CPT)], dst_v)

    def g_start(i, b):
        pltpu.make_async_copy(h_hbm.at[src_v.at[i]], rows[b], gsem[b]).start()

    def g_wait(i, b):
        pltpu.make_async_copy(h_hbm.at[src_v.at[i]], rows[b], gsem[b]).wait()

    def s_start(i, b):
        pltpu.make_async_copy(rows[b], acc_sh.at[dst_v.at[i]],
                              ssem[b]).start(add=True)

    def s_wait(i, b):
        pltpu.make_async_copy(rows[b], acc_sh.at[dst_v.at[i]], ssem[b]).wait()

    g_start(0, 0)
    g_start(1, 1)
    g_wait(0, 0); s_start(0, 0); g_start(2, 2)
    g_wait(1, 1); s_start(1, 1); g_start(3, 3)

    def group(g, carry):
        i0 = 4 * g + 2
        for k in range(4):
            i = i0 + k
            b = (2 + k) % 4
            bn = (b + 2) % 4
            g_wait(i, b)
            s_start(i, b)
            s_wait(i - 2, bn)
            g_start(i + 2, bn)
        return carry

    lax.fori_loop(0, (CPT - 4) // 4, group, 0)

    g_wait(CPT - 2, 2); s_start(CPT - 2, 2); s_wait(CPT - 4, 0)
    g_wait(CPT - 1, 3); s_start(CPT - 1, 3); s_wait(CPT - 3, 1)
    s_wait(CPT - 2, 2)
    s_wait(CPT - 1, 3)
    plsc.subcore_barrier()

    rr = sid * DR

    @pl.when(sid < NS - 1)
    def _():
        pltpu.sync_copy(acc_sh.at[pl.ds(rr, DR)],
                        out_hbm.at[pl.ds(cid * N + rr, DR)])

    @pl.when(sid == NS - 1)
    def _():
        pltpu.sync_copy(acc_sh.at[pl.ds(15 * DR, DR_LAST)],
                        out_hbm.at[pl.ds(cid * N + 15 * DR, DR_LAST)])


def _segment_sum_sc(h_split, src2, dst_p, zeros):
    mesh = plsc.VectorSubcoreMesh(core_axis_name="c", subcore_axis_name="s")
    return pl.kernel(
        _seg_sum_kernel,
        out_type=jax.ShapeDtypeStruct((NC * N, DH), jnp.float32),
        mesh=mesh,
        compiler_params=pltpu.CompilerParams(use_tc_tiling_on_sc=False),
        scratch_types=(
            [pltpu.VMEM((CPT, CHUNK), jnp.int32),
             pltpu.VMEM((CPT, CHUNK), jnp.int32)]
            + [pltpu.VMEM((CHUNK, DH), jnp.float32) for _ in range(4)]
            + [pltpu.VMEM_SHARED((N_ACC, DH), jnp.float32)]
            + [pltpu.SemaphoreType.DMA for _ in range(8)]
        ),
    )(h_split, src2, dst_p, zeros)


def _mm_relu_kernel(x_ref, w_ref, b_ref, o_ref):
    c = pl.program_id(0)
    res = jnp.maximum(
        jnp.dot(x_ref[...], w_ref[...], preferred_element_type=jnp.float32,
                precision=jax.lax.Precision.HIGHEST)
        + b_ref[...], 0.0)
    @pl.when(c == 0)
    def _():
        o_ref[...] = res[:, :DH]

    @pl.when(c == 1)
    def _():
        o_ref[...] = res[:, DH:]


def _gin_mm_kernel(hlo_ref, hhi_ref, plo_ref, phi_ref, w_ref, b_ref, o_ref):
    c = pl.program_id(0)
    t = jnp.concatenate([hlo_ref[...] + plo_ref[...],
                         hhi_ref[...] + phi_ref[...]], axis=1)
    res = jnp.maximum(
        jnp.dot(t, w_ref[...], preferred_element_type=jnp.float32,
                precision=jax.lax.Precision.HIGHEST)
        + b_ref[...], 0.0)
    @pl.when(c == 0)
    def _():
        o_ref[...] = res[:, :DH]

    @pl.when(c == 1)
    def _():
        o_ref[...] = res[:, DH:]


def _final_kernel(hlo_ref, hhi_ref, plo_ref, phi_ref, w_ref, b_ref,
                  wmt_ref, bm_ref, emb_ref, out_ref):
    t = jnp.concatenate([hlo_ref[...] + plo_ref[...],
                         hhi_ref[...] + phi_ref[...]], axis=1)
    h2 = jnp.maximum(
        jnp.dot(t, w_ref[...], preferred_element_type=jnp.float32,
                precision=jax.lax.Precision.HIGHEST)
        + b_ref[...], 0.0)
    emb_ref[...] = h2
    logits = jnp.sum(h2 * wmt_ref[...], axis=1, keepdims=True) + bm_ref[...]
    out_ref[...] = jax.nn.sigmoid(logits)


def _mm_relu_split(x, W, b):
    return pl.pallas_call(
        _mm_relu_kernel,
        grid=(NC, NBLK),
        in_specs=[pl.BlockSpec((RB, D), lambda c, i: (i, 0)),
                  pl.BlockSpec((D, D), lambda c, i: (0, 0)),
                  pl.BlockSpec((1, D), lambda c, i: (0, 0))],
        out_specs=pl.BlockSpec((RB, DH), lambda c, i: (c * NBLK + i, 0)),
        out_shape=jax.ShapeDtypeStruct((NC * N, DH), jnp.float32),
    )(x, W, b.reshape(1, D))


def _gin_mm_split(h_split, parts, W, b):
    return pl.pallas_call(
        _gin_mm_kernel,
        grid=(NC, NBLK),
        in_specs=[pl.BlockSpec((RB, DH), lambda c, i: (i, 0)),
                  pl.BlockSpec((RB, DH), lambda c, i: (NBLK + i, 0)),
                  pl.BlockSpec((RB, DH), lambda c, i: (i, 0)),
                  pl.BlockSpec((RB, DH), lambda c, i: (NBLK + i, 0)),
                  pl.BlockSpec((D, D), lambda c, i: (0, 0)),
                  pl.BlockSpec((1, D), lambda c, i: (0, 0))],
        out_specs=pl.BlockSpec((RB, DH), lambda c, i: (c * NBLK + i, 0)),
        out_shape=jax.ShapeDtypeStruct((NC * N, DH), jnp.float32),
    )(h_split, h_split, parts, parts, W, b.reshape(1, D))


def _final_mm(h_split, parts, W, b, Wm_t, bm):
    return pl.pallas_call(
        _final_kernel,
        grid=(NBLK,),
        in_specs=[pl.BlockSpec((RB, DH), lambda i: (i, 0)),
                  pl.BlockSpec((RB, DH), lambda i: (NBLK + i, 0)),
                  pl.BlockSpec((RB, DH), lambda i: (i, 0)),
                  pl.BlockSpec((RB, DH), lambda i: (NBLK + i, 0)),
                  pl.BlockSpec((D, D), lambda i: (0, 0)),
                  pl.BlockSpec((1, D), lambda i: (0, 0)),
                  pl.BlockSpec((1, D), lambda i: (0, 0)),
                  pl.BlockSpec((1, 1), lambda i: (0, 0))],
        out_specs=[pl.BlockSpec((RB, D), lambda i: (i, 0)),
                   pl.BlockSpec((RB, 1), lambda i: (i, 0))],
        out_shape=[jax.ShapeDtypeStruct((N, D), jnp.float32),
                   jax.ShapeDtypeStruct((N, 1), jnp.float32)],
    )(h_split, h_split, parts, parts, W, b.reshape(1, D), Wm_t,
      bm.reshape(1, 1))


def kernel(x, edge_index, Wf, bf, Wg0, bg0, gamma0, beta0,
           Wg1, bg1, gamma1, beta1, Wm, bm):
    src = edge_index[0]
    dst = edge_index[1]
    pad = E_PAD - E
    src_p = jnp.concatenate([src, jnp.zeros((pad,), jnp.int32)])
    src2 = jnp.concatenate([src_p, src_p + N]).reshape(NC * NS * CPT, CHUNK)
    dst_p = jnp.concatenate([dst, jnp.full((pad,), N, jnp.int32)])
    dst_p = dst_p.reshape(NS * CPT, CHUNK)
    zeros = jnp.zeros((N_ACC, DH), jnp.float32)

    s = 1.0 / jnp.sqrt(jnp.float32(1.0 + 1e-5))
    Wg0f = Wg0 * (gamma0 * s)[None, :]
    bg0f = bg0 * gamma0 * s + beta0
    Wg1f = Wg1 * (gamma1 * s)[None, :]
    bg1f = bg1 * gamma1 * s + beta1

    h0 = _mm_relu_split(x, Wf, bf)
    p0 = _segment_sum_sc(h0, src2, dst_p, zeros)
    h1 = _gin_mm_split(h0, p0, Wg0f, bg0f)
    p1 = _segment_sum_sc(h1, src2, dst_p, zeros)
    emb, out = _final_mm(h1, p1, Wg1f, bg1f, Wm.reshape(1, D), bm)
    return (out, emb)


# speedup vs baseline: 1.3543x; 1.3543x over previous
import jax
import jax.numpy as jnp
from jax import lax
from jax.experimental import pallas as pl
from jax.experimental.pallas import tpu as pltpu
from jax.experimental.pallas import tpu_sc as plsc

N = 10000
E = 320000
D = 128
DH = D // 2

NC = 2
NS = 16
CHUNK = 128
CPT = 160
E_PAD = NS * CHUNK * CPT
N_ACC = 10240
ZR = N_ACC // NS
DR = 632
DR_LAST = N - 15 * DR

RB = 2000
NBLK = N // RB


def _seg_sum_kernel(h_hbm, src_hbm, dst_hbm, z_hbm, out_hbm,
                    src_v, dst_v, r0, r1, r2, r3, acc_sh, h_sh,
                    g0, g1, g2, g3, s0, s1, s2, s3):
    rows = [r0, r1, r2, r3]
    gsem = [g0, g1, g2, g3]
    ssem = [s0, s1, s2, s3]
    cid = lax.axis_index("c")
    sid = lax.axis_index("s")

    pltpu.sync_copy(z_hbm.at[pl.ds(sid * ZR, ZR)], acc_sh.at[pl.ds(sid * ZR, ZR)])
    plsc.subcore_barrier()

    hr = sid * DR

    @pl.when(sid < NS - 1)
    def _():
        pltpu.sync_copy(h_hbm.at[pl.ds(cid * N + hr, DR)], h_sh.at[pl.ds(hr, DR)])

    @pl.when(sid == NS - 1)
    def _():
        pltpu.sync_copy(h_hbm.at[pl.ds(cid * N + 15 * DR, DR_LAST)],
                        h_sh.at[pl.ds(15 * DR, DR_LAST)])

    pltpu.sync_copy(src_hbm.at[pl.ds(sid * CPT, CPT)], src_v)
    pltpu.sync_copy(dst_hbm.at[pl.ds(sid * CPT, CPT)], dst_v)
    plsc.subcore_barrier()

    def step(i, carry):
        pltpu.async_copy(h_sh.at[src_v.at[i]], r0, g0).wait()
        pltpu.sync_copy(r0, acc_sh.at[dst_v.at[i]], add=True)
        return carry

    _ = (rows, gsem, r1, r2, r3, g1, g2, g3, s0, s1, s2, s3)
    lax.fori_loop(0, CPT, step, 0)
    plsc.subcore_barrier()

    rr = sid * DR

    @pl.when(sid < NS - 1)
    def _():
        pltpu.sync_copy(acc_sh.at[pl.ds(rr, DR)],
                        out_hbm.at[pl.ds(cid * N + rr, DR)])

    @pl.when(sid == NS - 1)
    def _():
        pltpu.sync_copy(acc_sh.at[pl.ds(15 * DR, DR_LAST)],
                        out_hbm.at[pl.ds(cid * N + 15 * DR, DR_LAST)])


def _segment_sum_sc(h_split, src2, dst_p, zeros):
    mesh = plsc.VectorSubcoreMesh(core_axis_name="c", subcore_axis_name="s")
    return pl.kernel(
        _seg_sum_kernel,
        out_type=jax.ShapeDtypeStruct((NC * N, DH), jnp.float32),
        mesh=mesh,
        compiler_params=pltpu.CompilerParams(use_tc_tiling_on_sc=False),
        scratch_types=(
            [pltpu.VMEM((CPT, CHUNK), jnp.int32),
             pltpu.VMEM((CPT, CHUNK), jnp.int32)]
            + [pltpu.VMEM((CHUNK, DH), jnp.float32) for _ in range(4)]
            + [pltpu.VMEM_SHARED((N_ACC, DH), jnp.float32),
               pltpu.VMEM_SHARED((N_ACC, DH), jnp.float32)]
            + [pltpu.SemaphoreType.DMA for _ in range(8)]
        ),
    )(h_split, src2, dst_p, zeros)


def _mm_relu_kernel(x_ref, w_ref, b_ref, o_ref):
    c = pl.program_id(0)
    res = jnp.maximum(
        jnp.dot(x_ref[...], w_ref[...], preferred_element_type=jnp.float32,
                precision=jax.lax.Precision.HIGHEST)
        + b_ref[...], 0.0)
    @pl.when(c == 0)
    def _():
        o_ref[...] = res[:, :DH]

    @pl.when(c == 1)
    def _():
        o_ref[...] = res[:, DH:]


def _gin_mm_kernel(hlo_ref, hhi_ref, plo_ref, phi_ref, w_ref, b_ref, o_ref):
    c = pl.program_id(0)
    t = jnp.concatenate([hlo_ref[...] + plo_ref[...],
                         hhi_ref[...] + phi_ref[...]], axis=1)
    res = jnp.maximum(
        jnp.dot(t, w_ref[...], preferred_element_type=jnp.float32,
                precision=jax.lax.Precision.HIGHEST)
        + b_ref[...], 0.0)
    @pl.when(c == 0)
    def _():
        o_ref[...] = res[:, :DH]

    @pl.when(c == 1)
    def _():
        o_ref[...] = res[:, DH:]


def _final_kernel(hlo_ref, hhi_ref, plo_ref, phi_ref, w_ref, b_ref,
                  wmt_ref, bm_ref, emb_ref, out_ref):
    t = jnp.concatenate([hlo_ref[...] + plo_ref[...],
                         hhi_ref[...] + phi_ref[...]], axis=1)
    h2 = jnp.maximum(
        jnp.dot(t, w_ref[...], preferred_element_type=jnp.float32,
                precision=jax.lax.Precision.HIGHEST)
        + b_ref[...], 0.0)
    emb_ref[...] = h2
    logits = jnp.sum(h2 * wmt_ref[...], axis=1, keepdims=True) + bm_ref[...]
    out_ref[...] = jax.nn.sigmoid(logits)


def _mm_relu_split(x, W, b):
    return pl.pallas_call(
        _mm_relu_kernel,
        grid=(NC, NBLK),
        in_specs=[pl.BlockSpec((RB, D), lambda c, i: (i, 0)),
                  pl.BlockSpec((D, D), lambda c, i: (0, 0)),
                  pl.BlockSpec((1, D), lambda c, i: (0, 0))],
        out_specs=pl.BlockSpec((RB, DH), lambda c, i: (c * NBLK + i, 0)),
        out_shape=jax.ShapeDtypeStruct((NC * N, DH), jnp.float32),
    )(x, W, b.reshape(1, D))


def _gin_mm_split(h_split, parts, W, b):
    return pl.pallas_call(
        _gin_mm_kernel,
        grid=(NC, NBLK),
        in_specs=[pl.BlockSpec((RB, DH), lambda c, i: (i, 0)),
                  pl.BlockSpec((RB, DH), lambda c, i: (NBLK + i, 0)),
                  pl.BlockSpec((RB, DH), lambda c, i: (i, 0)),
                  pl.BlockSpec((RB, DH), lambda c, i: (NBLK + i, 0)),
                  pl.BlockSpec((D, D), lambda c, i: (0, 0)),
                  pl.BlockSpec((1, D), lambda c, i: (0, 0))],
        out_specs=pl.BlockSpec((RB, DH), lambda c, i: (c * NBLK + i, 0)),
        out_shape=jax.ShapeDtypeStruct((NC * N, DH), jnp.float32),
    )(h_split, h_split, parts, parts, W, b.reshape(1, D))


def _final_mm(h_split, parts, W, b, Wm_t, bm):
    return pl.pallas_call(
        _final_kernel,
        grid=(NBLK,),
        in_specs=[pl.BlockSpec((RB, DH), lambda i: (i, 0)),
                  pl.BlockSpec((RB, DH), lambda i: (NBLK + i, 0)),
                  pl.BlockSpec((RB, DH), lambda i: (i, 0)),
                  pl.BlockSpec((RB, DH), lambda i: (NBLK + i, 0)),
                  pl.BlockSpec((D, D), lambda i: (0, 0)),
                  pl.BlockSpec((1, D), lambda i: (0, 0)),
                  pl.BlockSpec((1, D), lambda i: (0, 0)),
                  pl.BlockSpec((1, 1), lambda i: (0, 0))],
        out_specs=[pl.BlockSpec((RB, D), lambda i: (i, 0)),
                   pl.BlockSpec((RB, 1), lambda i: (i, 0))],
        out_shape=[jax.ShapeDtypeStruct((N, D), jnp.float32),
                   jax.ShapeDtypeStruct((N, 1), jnp.float32)],
    )(h_split, h_split, parts, parts, W, b.reshape(1, D), Wm_t,
      bm.reshape(1, 1))


def kernel(x, edge_index, Wf, bf, Wg0, bg0, gamma0, beta0,
           Wg1, bg1, gamma1, beta1, Wm, bm):
    src = edge_index[0]
    dst = edge_index[1]
    pad = E_PAD - E
    src_p = jnp.concatenate([src, jnp.zeros((pad,), jnp.int32)])
    src2 = src_p.reshape(NS * CPT, CHUNK)
    dst_p = jnp.concatenate([dst, jnp.full((pad,), N, jnp.int32)])
    dst_p = dst_p.reshape(NS * CPT, CHUNK)
    zeros = jnp.zeros((N_ACC, DH), jnp.float32)

    s = 1.0 / jnp.sqrt(jnp.float32(1.0 + 1e-5))
    Wg0f = Wg0 * (gamma0 * s)[None, :]
    bg0f = bg0 * gamma0 * s + beta0
    Wg1f = Wg1 * (gamma1 * s)[None, :]
    bg1f = bg1 * gamma1 * s + beta1

    h0 = _mm_relu_split(x, Wf, bf)
    p0 = _segment_sum_sc(h0, src2, dst_p, zeros)
    h1 = _gin_mm_split(h0, p0, Wg0f, bg0f)
    p1 = _segment_sum_sc(h1, src2, dst_p, zeros)
    emb, out = _final_mm(h1, p1, Wg1f, bg1f, Wm.reshape(1, D), bm)
    return (out, emb)
